# baseline (speedup 1.0000x reference)
import functools

import jax
import jax.numpy as jnp
from jax import lax
from jax.experimental import pallas as pl
from jax.experimental.pallas import tpu as pltpu
from jax.experimental.pallas import tpu_sc as plsc

_N = 10000
_E = 320000
_D = 128
_H = 64

_NC = 2
_NS = 16
_NW = _NC * _NS

_RPT = 640
_NPAD = _NS * _RPT
_CHUNK = 128
_EPT = 10240
_NCH = _EPT // _CHUNK
_EPAD = _NW * _EPT
_W128 = 128
_NRC = _RPT // _CHUNK

_mesh = plsc.VectorSubcoreMesh(
    core_axis_name="c", subcore_axis_name="s", num_cores=_NC, num_subcores=_NS
)


_K = 2
_NG = _NCH // _K


@functools.partial(
    pl.kernel,
    out_type=jax.ShapeDtypeStruct((_NC, _NPAD, _H), jnp.float32),
    mesh=_mesh,
    scratch_types=[
        pltpu.VMEM((_NCH, _CHUNK), jnp.int32),
        pltpu.VMEM((_NCH, _CHUNK), jnp.int32),
        pltpu.VMEM((_NRC, _CHUNK), jnp.int32),
        pltpu.VMEM((2, _K, _CHUNK, _H), jnp.float32),
        pltpu.VMEM_SHARED((_NPAD, _H), jnp.float32),
        pltpu.SemaphoreType.DMA,
        pltpu.SemaphoreType.DMA,
        pltpu.SemaphoreType.DMA,
        pltpu.SemaphoreType.DMA,
    ],
    compiler_params=pltpu.CompilerParams(use_tc_tiling_on_sc=False),
)
def _sc_agg(
    g_hbm, src_hbm, dst_hbm, rowidx_hbm, out_hbm, src_v, dst_v, row_v, bufs,
    acc_sh, gsem_a, gsem_b, ssem_a, ssem_b,
):
    c = lax.axis_index("c")
    s = lax.axis_index("s")
    wid = c * _NS + s
    pltpu.sync_copy(src_hbm.at[wid], src_v)
    pltpu.sync_copy(dst_hbm.at[wid], dst_v)
    pltpu.sync_copy(rowidx_hbm.at[c, s], row_v)
    r0 = s * _RPT
    for j in range(_NRC):
        pltpu.async_copy(g_hbm.at[row_v.at[j]], bufs.at[0, 0], gsem_a)
        pltpu.make_async_copy(g_hbm.at[row_v.at[j]], bufs.at[0, 0], gsem_a).wait()
        pltpu.sync_copy(bufs.at[0, 0], acc_sh.at[pl.ds(r0 + j * _CHUNK, _CHUNK)])
    plsc.subcore_barrier()

    def gathers(h, grp, sem):
        for k in range(_K):
            pltpu.async_copy(g_hbm.at[src_v.at[grp * _K + k]], bufs.at[h, k], sem)

    def wait_gathers(h, grp, sem):
        for k in range(_K):
            pltpu.make_async_copy(
                g_hbm.at[src_v.at[grp * _K + k]], bufs.at[h, k], sem
            ).wait()

    def scatters(h, grp, sem):
        for k in range(_K):
            pltpu.async_copy(
                bufs.at[h, k], acc_sh.at[dst_v.at[grp * _K + k]], sem, add=True
            )

    def wait_scatters(h, grp, sem):
        for k in range(_K):
            pltpu.make_async_copy(
                bufs.at[h, k], acc_sh.at[dst_v.at[grp * _K + k]], sem
            ).wait()

    gathers(0, 0, gsem_a)

    def body(j, carry):
        ga = 2 * j
        gb = 2 * j + 1
        wait_gathers(0, ga, gsem_a)
        scatters(0, ga, ssem_a)

        @pl.when(j > 0)
        def _():
            wait_scatters(1, gb - 2, ssem_b)

        gathers(1, gb, gsem_b)
        wait_gathers(1, gb, gsem_b)
        scatters(1, gb, ssem_b)

        @pl.when(j < _NG // 2 - 1)
        def _():
            wait_scatters(0, ga, ssem_a)
            gathers(0, ga + 2, gsem_a)

        return carry

    lax.fori_loop(0, _NG // 2, body, 0)
    wait_scatters(0, _NG - 2, ssem_a)
    wait_scatters(1, _NG - 1, ssem_b)
    plsc.subcore_barrier()
    pltpu.sync_copy(acc_sh.at[pl.ds(r0, _RPT)], out_hbm.at[c, pl.ds(r0, _RPT)])


_BLK = 512
_PBLK = _BLK // 2
_GRID = _NPAD // _BLK


def _unpair(p, v):
    stacked = jnp.concatenate([v[:, :_H], v[:, _H:]], axis=0)
    return jnp.dot(p, stacked, preferred_element_type=jnp.float32)


def _dis_blk(p, c0, c1, i):
    s = c0 + c1
    stacked = jnp.concatenate([s[:, 0:1], s[:, _H : _H + 1]], axis=0)
    deg = jnp.dot(p, stacked, preferred_element_type=jnp.float32)
    rows = i * _BLK + lax.broadcasted_iota(jnp.int32, (_BLK, 1), 0)
    return jnp.where(rows < _N, lax.rsqrt(deg), 0.0)


def _wide(v):
    return jnp.concatenate([v, jnp.zeros((_BLK, _W128 - _H), jnp.float32)], axis=1)


def _tc_prologue_body(p_ref, x_ref, w_ref, c0_ref, c1_ref, o_ref):
    i = pl.program_id(0)
    dis = _dis_blk(p_ref[...], c0_ref[...], c1_ref[...], i)
    o_ref[...] = _wide(
        dis * jnp.dot(x_ref[...], w_ref[...], preferred_element_type=jnp.float32)
    )


def _pspec():
    return pl.BlockSpec((_PBLK, _W128), lambda i: (i, 0))


def _permspec():
    return pl.BlockSpec((_BLK, _BLK), lambda i: (0, 0))


_tc_prologue = pl.pallas_call(
    _tc_prologue_body,
    out_shape=jax.ShapeDtypeStruct((_NPAD, _W128), jnp.float32),
    grid=(_GRID,),
    in_specs=[
        _permspec(),
        pl.BlockSpec((_BLK, _D), lambda i: (i, 0)),
        pl.BlockSpec((_D, _H), lambda i: (0, 0)),
        _pspec(),
        _pspec(),
    ],
    out_specs=pl.BlockSpec((_BLK, _W128), lambda i: (i, 0)),
)


def _tc_fuse_body(p_ref, a0_ref, a1_ref, c0_ref, c1_ref, b_ref, w_ref, o_ref):
    i = pl.program_id(0)
    dis = _dis_blk(p_ref[...], c0_ref[...], c1_ref[...], i)
    h = dis * _unpair(p_ref[...], a0_ref[...] + a1_ref[...]) + b_ref[...]
    h = jnp.maximum(h, 0.0)
    o_ref[...] = _wide(
        dis * jnp.dot(h, w_ref[...], preferred_element_type=jnp.float32)
    )


_tc_fuse = pl.pallas_call(
    _tc_fuse_body,
    out_shape=jax.ShapeDtypeStruct((_NPAD, _W128), jnp.float32),
    grid=(_GRID,),
    in_specs=[
        _permspec(),
        _pspec(),
        _pspec(),
        _pspec(),
        _pspec(),
        pl.BlockSpec((1, _H), lambda i: (0, 0)),
        pl.BlockSpec((_H, _H), lambda i: (0, 0)),
    ],
    out_specs=pl.BlockSpec((_BLK, _W128), lambda i: (i, 0)),
)


def _tc_final_body(
    p_ref, a0_ref, a1_ref, c0_ref, c1_ref, b_ref, w_ref, bo_ref, o_ref
):
    i = pl.program_id(0)
    dis = _dis_blk(p_ref[...], c0_ref[...], c1_ref[...], i)
    h = dis * _unpair(p_ref[...], a0_ref[...] + a1_ref[...]) + b_ref[...]
    o_ref[...] = (
        jnp.dot(h, w_ref[...], preferred_element_type=jnp.float32) + bo_ref[...]
    )


_tc_final = pl.pallas_call(
    _tc_final_body,
    out_shape=jax.ShapeDtypeStruct((_NPAD, _D), jnp.float32),
    grid=(_GRID,),
    in_specs=[
        _permspec(),
        _pspec(),
        _pspec(),
        _pspec(),
        _pspec(),
        pl.BlockSpec((1, _H), lambda i: (0, 0)),
        pl.BlockSpec((_H, _D), lambda i: (0, 0)),
        pl.BlockSpec((1, _D), lambda i: (0, 0)),
    ],
    out_specs=pl.BlockSpec((_BLK, _D), lambda i: (i, 0)),
)


def _agg(gwide, srcp, dstp, rowidx):
    acc = _sc_agg(gwide.reshape(2 * _NPAD, _H), srcp, dstp, rowidx)
    accp = acc.reshape(_NC, _NPAD // 2, _W128)
    return accp[0], accp[1]


def kernel(x, edge_index, W0, b0, W1, b1, W2, b2, W3, b3, Wout, bout):
    src = edge_index[0].astype(jnp.int32)
    dst = edge_index[1].astype(jnp.int32)
    npad = _EPAD - _E
    src1 = jnp.concatenate([src, jnp.full((npad,), _N, jnp.int32)])
    srcp = (2 * src1).reshape(_NW, _NCH, _CHUNK)
    dstp = jnp.concatenate([dst, jnp.full((npad,), _N, jnp.int32)]).reshape(
        _NW, _NCH, _CHUNK
    )
    nodes = jnp.arange(_NPAD, dtype=jnp.int32)
    rowidx = jnp.stack([2 * nodes, 2 * nodes + 1]).reshape(
        _NC, _NS, _NRC, _CHUNK
    )
    xp = jnp.pad(x, ((0, _NPAD - _N), (0, 0)))

    r = jnp.arange(_BLK)
    perm = (
        jnp.zeros((_BLK, _BLK), jnp.float32)
        .at[r, r // 2 + (r % 2) * _PBLK]
        .set(1.0)
    )

    ones_wide = jnp.concatenate(
        [jnp.ones((_NPAD, _H), jnp.float32), jnp.zeros((_NPAD, _H), jnp.float32)],
        axis=1,
    )
    c0, c1 = _agg(ones_wide, srcp, dstp, rowidx)

    g = _tc_prologue(perm, xp, W0, c0, c1)
    for bb, ww in ((b0, W1), (b1, W2), (b2, W3)):
        a0, a1 = _agg(g, srcp, dstp, rowidx)
        g = _tc_fuse(perm, a0, a1, c0, c1, bb.reshape(1, _H), ww)
    a0, a1 = _agg(g, srcp, dstp, rowidx)
    out = _tc_final(
        perm, a0, a1, c0, c1, b3.reshape(1, _H), Wout, bout.reshape(1, _D)
    )
    return out[:_N]
